# Initial kernel scaffold; baseline (speedup 1.0000x reference)
import jax
import jax.numpy as jnp
from jax.experimental import pallas as pl


def kernel(seq1, seq2, edge_index, msk, samp_bias1, samp_bias2, W_gcn, b_gcn, W_cat, b_cat, W_disc, b_disc):
    raise NotImplementedError("write your pallas kernel here")


# speedup vs baseline: 9.5717x; 9.5717x over previous
import functools

import jax
import jax.numpy as jnp
from jax import lax
from jax.experimental import pallas as pl
from jax.experimental.pallas import tpu as pltpu
from jax.experimental.pallas import tpu_sc as plsc

N = 10000
E = 160000
F = 256
NH = 256
NC = 2
NS = 16
NW = NC * NS
B = 128
EPAD = 163840
ROWS = 10240
RPT = ROWS // NS
DUMP = ROWS - N

_mesh = plsc.VectorSubcoreMesh(core_axis_name="c", subcore_axis_name="s")


@functools.partial(
    pl.kernel,
    out_type=jax.ShapeDtypeStruct((NC * ROWS, 16), jnp.float32),
    mesh=_mesh,
    scratch_types=[
        pltpu.VMEM((B,), jnp.int32),
        pltpu.VMEM((B, 16), jnp.float32),
        pltpu.VMEM_SHARED((ROWS, 16), jnp.float32),
    ],
)
def _deg_kernel(dst_hbm, ones_hbm, zeros_hbm, deg_out, dstb, onesb, acc):
    c = lax.axis_index("c")
    s = lax.axis_index("s")
    wid = c * NS + s
    pltpu.sync_copy(zeros_hbm, acc.at[pl.ds(s * RPT, RPT)])
    pltpu.sync_copy(ones_hbm, onesb)
    plsc.subcore_barrier()
    base = wid * (EPAD // NW)

    def body(j, carry):
        pltpu.sync_copy(dst_hbm.at[pl.ds(base + j * B, B)], dstb)
        pltpu.sync_copy(onesb, acc.at[dstb], add=True)
        return carry

    lax.fori_loop(0, EPAD // NW // B, body, 0)
    plsc.subcore_barrier()
    pltpu.sync_copy(acc.at[pl.ds(s * RPT, RPT)],
                    deg_out.at[pl.ds(c * ROWS + s * RPT, RPT)])


@functools.partial(
    pl.kernel,
    out_type=jax.ShapeDtypeStruct((4 * ROWS, 128), jnp.float32),
    mesh=_mesh,
    scratch_types=[
        pltpu.VMEM((B,), jnp.int32),
        pltpu.VMEM((B,), jnp.int32),
        pltpu.VMEM((B, 128), jnp.float32),
        pltpu.VMEM_SHARED((ROWS, 128), jnp.float32),
        pltpu.SemaphoreType.DMA,
    ],
)
def _scatter_kernel(hs_hbm, src4_hbm, dst_hbm, zeros_hbm, s_out,
                    srcb, dstb, rows, acc, sem):
    c = lax.axis_index("c")
    s = lax.axis_index("s")
    ept = EPAD // NS
    base = s * ept
    for k in range(2):
        chunk = c * 2 + k
        pltpu.sync_copy(zeros_hbm, acc.at[pl.ds(s * RPT, RPT)])
        plsc.subcore_barrier()

        def body(j, carry):
            off = base + j * B
            pltpu.sync_copy(src4_hbm.at[pl.ds(chunk * EPAD + off, B)], srcb)
            pltpu.sync_copy(dst_hbm.at[pl.ds(off, B)], dstb)
            pltpu.async_copy(hs_hbm.at[srcb], rows, sem).wait()
            pltpu.sync_copy(rows, acc.at[dstb], add=True)
            return carry

        lax.fori_loop(0, ept // B, body, 0)
        plsc.subcore_barrier()
        pltpu.sync_copy(acc.at[pl.ds(s * RPT, RPT)],
                        s_out.at[pl.ds(chunk * ROWS + s * RPT, RPT)])


def _mm_body(x_ref, w_ref, deg_ref, hs_ref):
    deg = deg_ref[0, :, 0] + deg_ref[1, :, 0] + 1.0
    dinv = lax.rsqrt(deg)
    h = jnp.dot(x_ref[...], w_ref[...], preferred_element_type=jnp.float32)
    hsv = h * dinv[:, None]
    hs_ref[0] = hsv[:, :128]
    hs_ref[1] = hsv[:, 128:]


def _post_body(sa, sb, ha, hb, da, db, mskr, bg, wcat, bc,
               h1c_ref, h2c_ref, cs_ref):
    dinv_a = lax.rsqrt(da[0, :, 0] + da[1, :, 0] + 1.0)[:, None]
    dinv_b = lax.rsqrt(db[0, :, 0] + db[1, :, 0] + 1.0)[:, None]
    bgv = bg[...][None, :]
    mskv = mskr[0, 0][:, None]

    def gcn(sref, href, i0, dinv):
        sv = jnp.concatenate([sref[i0], sref[i0 + 1]], axis=1)
        hv = jnp.concatenate([href[i0], href[i0 + 1]], axis=1)
        return jnp.maximum(dinv * (sv + hv) + bgv, 0.0)

    h1t = gcn(sa, ha, 0, dinv_a)
    h1b = gcn(sb, hb, 0, dinv_b)
    h2t = gcn(sa, ha, 2, dinv_a)
    h2b = gcn(sb, hb, 2, dinv_b)
    z1 = jnp.concatenate([h1t, h1b], axis=1)
    z2 = jnp.concatenate([h2t, h2b], axis=1)
    w = wcat[...]
    bcv = bc[...][None, :]
    h1c = jnp.maximum(jnp.dot(z1, w, preferred_element_type=jnp.float32) + bcv, 0.0)
    h2c = jnp.maximum(jnp.dot(z2, w, preferred_element_type=jnp.float32) + bcv, 0.0)
    h1c_ref[...] = h1c
    h2c_ref[...] = h2c
    cs_ref[0, 0] = jnp.sum(h1c * mskv, axis=0)


def _disc_body(cs, mskr, wd, h1c, h2c, sb1, sb2, bd, out_ref):
    msum = jnp.sum(mskr[...])
    cvec = jnp.sum(cs[...], axis=(0, 1)) / msum
    cvec = 1.0 / (1.0 + jnp.exp(-cvec))
    wcv = jnp.dot(wd[...], cvec[:, None], preferred_element_type=jnp.float32)
    s1 = jnp.dot(h1c[...], wcv, preferred_element_type=jnp.float32)[:, 0]
    s2 = jnp.dot(h2c[...], wcv, preferred_element_type=jnp.float32)[:, 0]
    out_ref[pl.ds(0, N // 2)] = s1 + bd[0] + sb1[...]
    out_ref[pl.ds(N // 2, N // 2)] = s2 + bd[0] + sb2[...]


def kernel(seq1, seq2, edge_index, msk, samp_bias1, samp_bias2,
           W_gcn, b_gcn, W_cat, b_cat, W_disc, b_disc):
    src = edge_index[0]
    dst = edge_index[1]
    pad = EPAD - E
    pad_src = (jnp.arange(pad, dtype=jnp.int32) * 79) % N
    pad_dst = N + (jnp.arange(pad, dtype=jnp.int32) % DUMP)
    src_p = jnp.concatenate([src, pad_src])
    dst_p = jnp.concatenate([dst, pad_dst])
    offs = jnp.arange(4, dtype=jnp.int32) * N
    src4 = (src_p[None, :] + offs[:, None]).reshape(-1)

    ones16 = jnp.ones((B, 16), jnp.float32)
    zeros16 = jnp.zeros((RPT, 16), jnp.float32)
    zeros128 = jnp.zeros((RPT, 128), jnp.float32)

    deg2 = _deg_kernel(dst_p, ones16, zeros16)
    deg3 = deg2.reshape(NC, ROWS, 16)

    X2 = jnp.concatenate([seq1, seq2], axis=0)
    hs = pl.pallas_call(
        _mm_body,
        grid=(10,),
        in_specs=[
            pl.BlockSpec((2000, F), lambda i: (i, 0)),
            pl.BlockSpec((F, NH), lambda i: (0, 0)),
            pl.BlockSpec((NC, 2000, 16), lambda i: (0, i % 5, 0)),
        ],
        out_specs=pl.BlockSpec((2, 2000, 128), lambda i: (i // 5, i % 5, 0)),
        out_shape=jax.ShapeDtypeStruct((4, N, 128), jnp.float32),
    )(X2, W_gcn, deg3)

    s_flat = _scatter_kernel(hs.reshape(4 * N, 128), src4, dst_p, zeros128)
    S = s_flat.reshape(4, ROWS, 128)

    half = N // 2
    msk3 = msk.reshape(5, 1, 1000)
    h1c, h2c, cs = pl.pallas_call(
        _post_body,
        grid=(5,),
        in_specs=[
            pl.BlockSpec((4, 1000, 128), lambda i: (0, i, 0)),
            pl.BlockSpec((4, 1000, 128), lambda i: (0, i + 5, 0)),
            pl.BlockSpec((4, 1000, 128), lambda i: (0, i, 0)),
            pl.BlockSpec((4, 1000, 128), lambda i: (0, i + 5, 0)),
            pl.BlockSpec((NC, 1000, 16), lambda i: (0, i, 0)),
            pl.BlockSpec((NC, 1000, 16), lambda i: (0, i + 5, 0)),
            pl.BlockSpec((1, 1, 1000), lambda i: (i, 0, 0)),
            pl.BlockSpec((NH,), lambda i: (0,)),
            pl.BlockSpec((2 * NH, NH), lambda i: (0, 0)),
            pl.BlockSpec((NH,), lambda i: (0,)),
        ],
        out_specs=[
            pl.BlockSpec((1000, NH), lambda i: (i, 0)),
            pl.BlockSpec((1000, NH), lambda i: (i, 0)),
            pl.BlockSpec((1, 1, NH), lambda i: (i, 0, 0)),
        ],
        out_shape=[
            jax.ShapeDtypeStruct((half, NH), jnp.float32),
            jax.ShapeDtypeStruct((half, NH), jnp.float32),
            jax.ShapeDtypeStruct((5, 1, NH), jnp.float32),
        ],
    )(S, S, hs, hs, deg3, deg3, msk3, b_gcn, W_cat, b_cat)

    logits = pl.pallas_call(
        _disc_body,
        out_shape=jax.ShapeDtypeStruct((N,), jnp.float32),
    )(cs, msk, W_disc, h1c, h2c, samp_bias1, samp_bias2,
      b_disc.reshape(1))
    return logits
